# Initial kernel scaffold; baseline (speedup 1.0000x reference)
import functools

import jax
import jax.numpy as jnp
from jax import lax
from jax.experimental import pallas as pl
from jax.experimental.pallas import tpu as pltpu
from jax.experimental.pallas import tpu_sc as plsc

N = 10000
D = 128
E = 320000
G = 128
NC, NS = 2, 16
NW = NC * NS
C = 128
EPT = 10112
NCHUNK = EPT // C
EP = EPT * NW
ACC_ROWS = 10240
ZCHUNKS = ACC_ROWS // NS // 16
RPT = N // NS
COPC = 125
RB = 1000
NG = N // RB

_mesh = plsc.VectorSubcoreMesh(core_axis_name="c", subcore_axis_name="s")


@functools.partial(
    pl.kernel,
    out_type=jax.ShapeDtypeStruct((NC, N, 16), jnp.float32),
    mesh=_mesh,
    scratch_types=[
        pltpu.VMEM((C,), jnp.int32),
        pltpu.VMEM((C, 16), jnp.float32),
        pltpu.VMEM((ACC_ROWS // NS, 16), jnp.float32),
        pltpu.VMEM((RPT, 16), jnp.float32),
        pltpu.VMEM_SHARED((ACC_ROWS, 16), jnp.float32),
    ],
)
def _deg_sc(dst_hbm, out_hbm, didx, ones_v, zbuf, obuf, acc):
    c = lax.axis_index("c")
    s = lax.axis_index("s")
    w = c * NS + s

    def _fill(i, _):
        zbuf[i, :] = jnp.zeros((16,), jnp.float32)
        return 0

    def _fill_ones(i, _):
        ones_v[i, :] = jnp.ones((16,), jnp.float32)
        return 0

    lax.fori_loop(0, ACC_ROWS // NS, _fill, 0)
    lax.fori_loop(0, C, _fill_ones, 0)
    pltpu.sync_copy(zbuf, acc.at[pl.ds(s * (ACC_ROWS // NS), ACC_ROWS // NS)])
    plsc.subcore_barrier()

    def _edges(k, _):
        off = pl.multiple_of(w * EPT + k * C, 8)
        pltpu.sync_copy(dst_hbm.at[pl.ds(off, C)], didx)
        pltpu.sync_copy(ones_v, acc.at[didx], add=True)
        return 0

    lax.fori_loop(0, NCHUNK, _edges, 0)
    plsc.subcore_barrier()
    pltpu.sync_copy(acc.at[pl.ds(s * RPT, RPT)], obuf)
    pltpu.sync_copy(obuf, out_hbm.at[c, pl.ds(s * RPT, RPT)])


@functools.partial(
    pl.kernel,
    out_type=jax.ShapeDtypeStruct((NC, N, D), jnp.float32),
    mesh=_mesh,
    scratch_types=[
        pltpu.VMEM((C,), jnp.int32),
        pltpu.VMEM((C,), jnp.int32),
        pltpu.VMEM((C, D), jnp.float32),
        pltpu.VMEM((16, D), jnp.float32),
        pltpu.VMEM((COPC, D), jnp.float32),
        pltpu.VMEM_SHARED((ACC_ROWS, D), jnp.float32),
        pltpu.SemaphoreType.DMA,
    ],
)
def _edge_sc(hs_hbm, src_hbm, dst_hbm, out_hbm,
             sidx, didx, rows, zrow, obuf, acc, sem):
    c = lax.axis_index("c")
    s = lax.axis_index("s")
    w = c * NS + s

    def _zf(i, _):
        r = i // 8
        q = (i % 8) * 16
        zrow[r, pl.ds(q, 16)] = jnp.zeros((16,), jnp.float32)
        return 0

    lax.fori_loop(0, 128, _zf, 0)

    def _zacc(j, _):
        pltpu.sync_copy(zrow, acc.at[pl.ds(s * (ACC_ROWS // NS) + j * 16, 16)])
        return 0

    lax.fori_loop(0, ZCHUNKS, _zacc, 0)
    plsc.subcore_barrier()

    def _edges(k, _):
        off = pl.multiple_of(w * EPT + k * C, 8)
        pltpu.sync_copy(src_hbm.at[pl.ds(off, C)], sidx)
        pltpu.sync_copy(dst_hbm.at[pl.ds(off, C)], didx)
        pltpu.async_copy(hs_hbm.at[sidx], rows, sem).wait()
        pltpu.sync_copy(rows, acc.at[didx], add=True)
        return 0

    lax.fori_loop(0, NCHUNK, _edges, 0)
    plsc.subcore_barrier()

    def _co(j, _):
        r0 = s * RPT + j * COPC
        pltpu.sync_copy(acc.at[pl.ds(r0, COPC)], obuf)
        pltpu.sync_copy(obuf, out_hbm.at[c, pl.ds(r0, COPC)])
        return 0

    lax.fori_loop(0, RPT // COPC, _co, 0)


def _tc1_body(x_ref, w_ref, degp_ref, hs_ref, dinv_ref):
    deg = degp_ref[0, :, 0:1] + degp_ref[1, :, 0:1] + 1.0
    dinv = lax.rsqrt(deg)
    h = jnp.dot(x_ref[...], w_ref[...], preferred_element_type=jnp.float32)
    hs_ref[...] = h * dinv
    dinv_ref[...] = dinv


def _tc1(x, W1, degp):
    return pl.pallas_call(
        _tc1_body,
        grid=(NG,),
        in_specs=[
            pl.BlockSpec((RB, D), lambda i: (i, 0)),
            pl.BlockSpec((D, D), lambda i: (0, 0)),
            pl.BlockSpec((NC, RB, 16), lambda i: (0, i, 0)),
        ],
        out_specs=[
            pl.BlockSpec((RB, D), lambda i: (i, 0)),
            pl.BlockSpec((RB, 1), lambda i: (i, 0)),
        ],
        out_shape=[
            jax.ShapeDtypeStruct((N, D), jnp.float32),
            jax.ShapeDtypeStruct((N, 1), jnp.float32),
        ],
    )(x, W1, degp)


def _tcmid_body(p_ref, hs_ref, dinv_ref, b_ref, w_ref, out_ref):
    dinv = dinv_ref[...]
    pre = (p_ref[0] + p_ref[1] + hs_ref[...]) * dinv + b_ref[...]
    f = jnp.maximum(pre, 0.0)
    out_ref[...] = jnp.dot(f, w_ref[...], preferred_element_type=jnp.float32) * dinv


def _tcmid(p, hs, dinv, b, Wn):
    return pl.pallas_call(
        _tcmid_body,
        grid=(NG,),
        in_specs=[
            pl.BlockSpec((NC, RB, D), lambda i: (0, i, 0)),
            pl.BlockSpec((RB, D), lambda i: (i, 0)),
            pl.BlockSpec((RB, 1), lambda i: (i, 0)),
            pl.BlockSpec((1, D), lambda i: (0, 0)),
            pl.BlockSpec((D, D), lambda i: (0, 0)),
        ],
        out_specs=pl.BlockSpec((RB, D), lambda i: (i, 0)),
        out_shape=jax.ShapeDtypeStruct((N, D), jnp.float32),
    )(p, hs, dinv, b, Wn)


def _tcfin_body(p_ref, hs_ref, dinv_ref, b_ref, bat_ref, lw_ref, lb_ref,
                out_ref, sum_acc, cnt_acc):
    i = pl.program_id(0)

    @pl.when(i == 0)
    def _():
        sum_acc[...] = jnp.zeros_like(sum_acc)
        cnt_acc[...] = jnp.zeros_like(cnt_acc)

    h3 = (p_ref[0] + p_ref[1] + hs_ref[...]) * dinv_ref[...] + b_ref[...]
    gids = lax.broadcasted_iota(jnp.int32, (RB, G), 1)
    oh = (gids == bat_ref[...]).astype(jnp.float32)
    dn = (((0,), (0,)), ((), ()))
    sum_acc[...] += lax.dot_general(oh, h3, dn,
                                    preferred_element_type=jnp.float32)
    cnt_acc[...] += lax.dot_general(oh, jnp.ones((RB, 1), jnp.float32), dn,
                                    preferred_element_type=jnp.float32)

    @pl.when(i == NG - 1)
    def _():
        pooled = sum_acc[...] / jnp.maximum(cnt_acc[...], 1.0)
        out_ref[...] = jnp.dot(pooled, lw_ref[...],
                               preferred_element_type=jnp.float32) + lb_ref[...]


def _tcfin(p, hs, dinv, b, batcol, linWp, linbp):
    return pl.pallas_call(
        _tcfin_body,
        grid=(NG,),
        in_specs=[
            pl.BlockSpec((NC, RB, D), lambda i: (0, i, 0)),
            pl.BlockSpec((RB, D), lambda i: (i, 0)),
            pl.BlockSpec((RB, 1), lambda i: (i, 0)),
            pl.BlockSpec((1, D), lambda i: (0, 0)),
            pl.BlockSpec((RB, 1), lambda i: (i, 0)),
            pl.BlockSpec((D, D), lambda i: (0, 0)),
            pl.BlockSpec((1, D), lambda i: (0, 0)),
        ],
        out_specs=pl.BlockSpec((G, D), lambda i: (0, 0)),
        out_shape=jax.ShapeDtypeStruct((G, D), jnp.float32),
        scratch_shapes=[
            pltpu.VMEM((G, D), jnp.float32),
            pltpu.VMEM((G, 1), jnp.float32),
        ],
    )(p, hs, dinv, b, batcol, linWp, linbp)


def kernel(x, edge_index, batch, W1, b1, W2, b2, W3, b3, linW, linb):
    ei = edge_index.astype(jnp.int32)
    pad = EP - E
    src = jnp.concatenate([ei[0], jnp.zeros((pad,), jnp.int32)])
    dst = jnp.concatenate([ei[1], jnp.full((pad,), N, jnp.int32)])
    batcol = batch.astype(jnp.int32).reshape(N, 1)
    b1r = b1.reshape(1, D)
    b2r = b2.reshape(1, D)
    b3r = b3.reshape(1, D)
    linWp = jnp.pad(linW, ((0, 0), (0, D - linW.shape[1])))
    linbp = jnp.pad(linb, (0, D - linb.shape[0])).reshape(1, D)

    degp = _deg_sc(dst)
    hs1, dinv = _tc1(x, W1, degp)
    p1 = _edge_sc(hs1, src, dst)
    hs2 = _tcmid(p1, hs1, dinv, b1r, W2)
    p2 = _edge_sc(hs2, src, dst)
    hs3 = _tcmid(p2, hs2, dinv, b2r, W3)
    p3 = _edge_sc(hs3, src, dst)
    outp = _tcfin(p3, hs3, dinv, b3r, batcol, linWp, linbp)
    return outp[:, :linW.shape[1]]


# speedup vs baseline: 9.2402x; 9.2402x over previous
import functools

import jax
import jax.numpy as jnp
from jax import lax
from jax.experimental import pallas as pl
from jax.experimental.pallas import tpu as pltpu
from jax.experimental.pallas import tpu_sc as plsc

N = 10000
D = 128
E = 320000
G = 128
NC, NS = 2, 16
NW = NC * NS
C = 128
EPT = 10112
NCHUNK = EPT // C
EP = EPT * NW
ACC_ROWS = 10240
ZCHUNKS = ACC_ROWS // NS // 16
RPT = ACC_ROWS // NS
COPC = 128
RB = 1000
NG = N // RB

_mesh = plsc.VectorSubcoreMesh(core_axis_name="c", subcore_axis_name="s")


@functools.partial(
    pl.kernel,
    out_type=jax.ShapeDtypeStruct((NC, ACC_ROWS, D), jnp.float32),
    mesh=_mesh,
    scratch_types=[
        pltpu.VMEM((C,), jnp.int32),
        pltpu.VMEM((C, D), jnp.float32),
        pltpu.VMEM((16, D), jnp.float32),
        pltpu.VMEM((COPC, D), jnp.float32),
        pltpu.VMEM_SHARED((ACC_ROWS, D), jnp.float32),
    ],
)
def _deg_sc(dst_hbm, out_hbm, didx, ones_v, zrow, obuf, acc):
    c = lax.axis_index("c")
    s = lax.axis_index("s")
    w = c * NS + s

    def _zf(i, _):
        r = i // 8
        q = (i % 8) * 16
        zrow[r, pl.ds(q, 16)] = jnp.zeros((16,), jnp.float32)
        return 0

    lax.fori_loop(0, 128, _zf, 0)

    def _of(i, _):
        r = i // 8
        q = (i % 8) * 16
        ones_v[r, pl.ds(q, 16)] = jnp.ones((16,), jnp.float32)
        return 0

    lax.fori_loop(0, C * 8, _of, 0)

    def _zacc(j, _):
        pltpu.sync_copy(zrow, acc.at[pl.ds(s * RPT + j * 16, 16)])
        return 0

    lax.fori_loop(0, ZCHUNKS, _zacc, 0)
    plsc.subcore_barrier()

    def _edges(k, _):
        off = pl.multiple_of(w * EPT + k * C, 8)
        pltpu.sync_copy(dst_hbm.at[pl.ds(off, C)], didx)
        pltpu.sync_copy(ones_v, acc.at[didx], add=True)
        return 0

    lax.fori_loop(0, NCHUNK, _edges, 0)
    plsc.subcore_barrier()

    def _co(j, _):
        r0 = s * RPT + j * COPC
        pltpu.sync_copy(acc.at[pl.ds(r0, COPC)], obuf)
        pltpu.sync_copy(obuf, out_hbm.at[c, pl.ds(r0, COPC)])
        return 0

    lax.fori_loop(0, RPT // COPC, _co, 0)


@functools.partial(
    pl.kernel,
    out_type=jax.ShapeDtypeStruct((NC, ACC_ROWS, D), jnp.float32),
    mesh=_mesh,
    scratch_types=[
        pltpu.VMEM((C,), jnp.int32),
        pltpu.VMEM((C,), jnp.int32),
        pltpu.VMEM((C, D), jnp.float32),
        pltpu.VMEM((16, D), jnp.float32),
        pltpu.VMEM((COPC, D), jnp.float32),
        pltpu.VMEM_SHARED((ACC_ROWS, D), jnp.float32),
        pltpu.SemaphoreType.DMA,
    ],
)
def _edge_sc(hs_hbm, src_hbm, dst_hbm, out_hbm,
             sidx, didx, rows, zrow, obuf, acc, sem):
    c = lax.axis_index("c")
    s = lax.axis_index("s")
    w = c * NS + s

    def _zf(i, _):
        r = i // 8
        q = (i % 8) * 16
        zrow[r, pl.ds(q, 16)] = jnp.zeros((16,), jnp.float32)
        return 0

    lax.fori_loop(0, 128, _zf, 0)

    def _zacc(j, _):
        pltpu.sync_copy(zrow, acc.at[pl.ds(s * (ACC_ROWS // NS) + j * 16, 16)])
        return 0

    lax.fori_loop(0, ZCHUNKS, _zacc, 0)
    plsc.subcore_barrier()

    def _edges(k, _):
        off = pl.multiple_of(w * EPT + k * C, 8)
        pltpu.sync_copy(src_hbm.at[pl.ds(off, C)], sidx)
        pltpu.sync_copy(dst_hbm.at[pl.ds(off, C)], didx)
        pltpu.async_copy(hs_hbm.at[sidx], rows, sem).wait()
        pltpu.sync_copy(rows, acc.at[didx], add=True)
        return 0

    lax.fori_loop(0, NCHUNK, _edges, 0)
    plsc.subcore_barrier()

    def _co(j, _):
        r0 = s * RPT + j * COPC
        pltpu.sync_copy(acc.at[pl.ds(r0, COPC)], obuf)
        pltpu.sync_copy(obuf, out_hbm.at[c, pl.ds(r0, COPC)])
        return 0

    lax.fori_loop(0, RPT // COPC, _co, 0)


def _tc1_body(x_ref, w_ref, degp_ref, hs_ref, dinv_ref):
    deg = degp_ref[0, :, 0:1] + degp_ref[1, :, 0:1] + 1.0
    dinv = lax.rsqrt(deg)
    h = jnp.dot(x_ref[...], w_ref[...], preferred_element_type=jnp.float32)
    hs_ref[...] = h * dinv
    dinv_ref[...] = dinv


def _tc1(x, W1, degp):
    return pl.pallas_call(
        _tc1_body,
        grid=(NG,),
        in_specs=[
            pl.BlockSpec((RB, D), lambda i: (i, 0)),
            pl.BlockSpec((D, D), lambda i: (0, 0)),
            pl.BlockSpec((NC, RB, D), lambda i: (0, i, 0)),
        ],
        out_specs=[
            pl.BlockSpec((RB, D), lambda i: (i, 0)),
            pl.BlockSpec((RB, 1), lambda i: (i, 0)),
        ],
        out_shape=[
            jax.ShapeDtypeStruct((N, D), jnp.float32),
            jax.ShapeDtypeStruct((N, 1), jnp.float32),
        ],
    )(x, W1, degp)


def _tcmid_body(p_ref, hs_ref, dinv_ref, b_ref, w_ref, out_ref):
    dinv = dinv_ref[...]
    pre = (p_ref[0] + p_ref[1] + hs_ref[...]) * dinv + b_ref[...]
    f = jnp.maximum(pre, 0.0)
    out_ref[...] = jnp.dot(f, w_ref[...], preferred_element_type=jnp.float32) * dinv


def _tcmid(p, hs, dinv, b, Wn):
    return pl.pallas_call(
        _tcmid_body,
        grid=(NG,),
        in_specs=[
            pl.BlockSpec((NC, RB, D), lambda i: (0, i, 0)),
            pl.BlockSpec((RB, D), lambda i: (i, 0)),
            pl.BlockSpec((RB, 1), lambda i: (i, 0)),
            pl.BlockSpec((1, D), lambda i: (0, 0)),
            pl.BlockSpec((D, D), lambda i: (0, 0)),
        ],
        out_specs=pl.BlockSpec((RB, D), lambda i: (i, 0)),
        out_shape=jax.ShapeDtypeStruct((N, D), jnp.float32),
    )(p, hs, dinv, b, Wn)


def _tcfin_body(p_ref, hs_ref, dinv_ref, b_ref, bat_ref, lw_ref, lb_ref,
                out_ref, sum_acc, cnt_acc):
    i = pl.program_id(0)

    @pl.when(i == 0)
    def _():
        sum_acc[...] = jnp.zeros_like(sum_acc)
        cnt_acc[...] = jnp.zeros_like(cnt_acc)

    h3 = (p_ref[0] + p_ref[1] + hs_ref[...]) * dinv_ref[...] + b_ref[...]
    gids = lax.broadcasted_iota(jnp.int32, (RB, G), 1)
    oh = (gids == bat_ref[...]).astype(jnp.float32)
    dn = (((0,), (0,)), ((), ()))
    sum_acc[...] += lax.dot_general(oh, h3, dn,
                                    preferred_element_type=jnp.float32)
    cnt_acc[...] += lax.dot_general(oh, jnp.ones((RB, 1), jnp.float32), dn,
                                    preferred_element_type=jnp.float32)

    @pl.when(i == NG - 1)
    def _():
        pooled = sum_acc[...] / jnp.maximum(cnt_acc[...], 1.0)
        out_ref[...] = jnp.dot(pooled, lw_ref[...],
                               preferred_element_type=jnp.float32) + lb_ref[...]


def _tcfin(p, hs, dinv, b, batcol, linWp, linbp):
    return pl.pallas_call(
        _tcfin_body,
        grid=(NG,),
        in_specs=[
            pl.BlockSpec((NC, RB, D), lambda i: (0, i, 0)),
            pl.BlockSpec((RB, D), lambda i: (i, 0)),
            pl.BlockSpec((RB, 1), lambda i: (i, 0)),
            pl.BlockSpec((1, D), lambda i: (0, 0)),
            pl.BlockSpec((RB, 1), lambda i: (i, 0)),
            pl.BlockSpec((D, D), lambda i: (0, 0)),
            pl.BlockSpec((1, D), lambda i: (0, 0)),
        ],
        out_specs=pl.BlockSpec((G, D), lambda i: (0, 0)),
        out_shape=jax.ShapeDtypeStruct((G, D), jnp.float32),
        scratch_shapes=[
            pltpu.VMEM((G, D), jnp.float32),
            pltpu.VMEM((G, 1), jnp.float32),
        ],
    )(p, hs, dinv, b, batcol, linWp, linbp)


def kernel(x, edge_index, batch, W1, b1, W2, b2, W3, b3, linW, linb):
    ei = edge_index.astype(jnp.int32)
    pad = EP - E
    src = jnp.concatenate([ei[0], jnp.zeros((pad,), jnp.int32)])
    dst = jnp.concatenate([ei[1], jnp.full((pad,), N, jnp.int32)])
    batcol = batch.astype(jnp.int32).reshape(N, 1)
    b1r = b1.reshape(1, D)
    b2r = b2.reshape(1, D)
    b3r = b3.reshape(1, D)
    linWp = jnp.pad(linW, ((0, 0), (0, D - linW.shape[1])))
    linbp = jnp.pad(linb, (0, D - linb.shape[0])).reshape(1, D)

    degp = _deg_sc(dst)
    hs1, dinv = _tc1(x, W1, degp)
    p1 = _edge_sc(hs1, src, dst)
    hs2 = _tcmid(p1, hs1, dinv, b1r, W2)
    p2 = _edge_sc(hs2, src, dst)
    hs3 = _tcmid(p2, hs2, dinv, b2r, W3)
    p3 = _edge_sc(hs3, src, dst)
    outp = _tcfin(p3, hs3, dinv, b3r, batcol, linWp, linbp)
    return outp[:, :linW.shape[1]]
